# baseline (speedup 1.0000x reference)
import jax
import jax.numpy as jnp
from jax.experimental import pallas as pl
from jax.experimental.pallas import tpu as pltpu

_BLOCK = 5000
_NBUF = 4


def _combine_kernel(wlt_ref, blt_ref, w1_ref, b1_ref, wc_ref, bc_ref):
    wc_ref[...] = jnp.dot(wlt_ref[...], w1_ref[...], preferred_element_type=jnp.float32)
    bc_ref[...] = jnp.dot(blt_ref[...], w1_ref[...], preferred_element_type=jnp.float32) + b1_ref[...]


def _mlp_pipe_kernel(f_hbm, wc_ref, bc_ref, w2_ref, b2_ref, o_hbm,
                     in_buf, out_buf, in_sem, out_sem):
    n = f_hbm.shape[0]
    nsteps = n // _BLOCK

    def in_copy(i, slot):
        return pltpu.make_async_copy(
            f_hbm.at[pl.ds(i * _BLOCK, _BLOCK), :], in_buf.at[slot], in_sem.at[slot])

    def out_copy(i, slot):
        return pltpu.make_async_copy(
            out_buf.at[slot], o_hbm.at[pl.ds(i * _BLOCK, _BLOCK), :], out_sem.at[slot])

    for j in range(_NBUF - 1):
        in_copy(j, j).start()

    def body(i, _):
        slot = jax.lax.rem(i, _NBUF)

        @pl.when(i + _NBUF - 1 < nsteps)
        def _():
            in_copy(i + _NBUF - 1, jax.lax.rem(i + _NBUF - 1, _NBUF)).start()

        in_copy(i, slot).wait()

        @pl.when(i >= _NBUF)
        def _():
            out_copy(i - _NBUF, slot).wait()

        f = in_buf[slot]
        h = jnp.dot(f, wc_ref[...], preferred_element_type=jnp.float32) + bc_ref[...]
        h = jnp.maximum(h, 0.0)
        out_buf[slot] = jnp.dot(h, w2_ref[...], preferred_element_type=jnp.float32) + b2_ref[...]
        out_copy(i, slot).start()
        return ()

    jax.lax.fori_loop(0, nsteps, body, ())

    for j in range(_NBUF):
        i = nsteps - _NBUF + j
        if i >= 0:
            out_copy(i, i % _NBUF).wait()


def kernel(features, points, nuv, ranges, W_lt, b_lt, W1, b1, W2, b2):
    del points, nuv, ranges
    n, d_in = features.shape
    d_out = W_lt.shape[0]
    wlt = W_lt.T
    w1 = W1.T
    w2 = W2.T
    blt = b_lt[None, :]
    b1r = b1[None, :]
    b2r = b2[None, :]

    wc, bc = pl.pallas_call(
        _combine_kernel,
        out_shape=(
            jax.ShapeDtypeStruct((d_in, d_out), jnp.float32),
            jax.ShapeDtypeStruct((1, d_out), jnp.float32),
        ),
    )(wlt, blt, w1, b1r)

    vmem = pl.BlockSpec(memory_space=pltpu.MemorySpace.VMEM)
    return pl.pallas_call(
        _mlp_pipe_kernel,
        in_specs=[
            pl.BlockSpec(memory_space=pl.ANY),
            vmem, vmem, vmem, vmem,
        ],
        out_specs=pl.BlockSpec(memory_space=pl.ANY),
        out_shape=jax.ShapeDtypeStruct((n, d_out), jnp.float32),
        scratch_shapes=[
            pltpu.VMEM((_NBUF, _BLOCK, d_out), jnp.float32),
            pltpu.VMEM((_NBUF, _BLOCK, d_out), jnp.float32),
            pltpu.SemaphoreType.DMA((_NBUF,)),
            pltpu.SemaphoreType.DMA((_NBUF,)),
        ],
    )(features, wc, bc, w2, b2r)


# speedup vs baseline: 1.2218x; 1.2218x over previous
import jax
import jax.numpy as jnp
from jax.experimental import pallas as pl
from jax.experimental.pallas import tpu as pltpu

_BLOCK = 20000


def _mlp_kernel(f_ref, wlt_ref, blt_ref, w1_ref, b1_ref, w2_ref, b2_ref, o_ref):
    wc = jax.lax.dot_general(
        wlt_ref[...], w1_ref[...], (((0,), (1,)), ((), ())),
        preferred_element_type=jnp.float32)
    bc = jax.lax.dot_general(
        blt_ref[...], w1_ref[...], (((1,), (1,)), ((), ())),
        preferred_element_type=jnp.float32) + b1_ref[...]
    f = f_ref[...]
    h = jnp.dot(f, wc, preferred_element_type=jnp.float32) + bc
    h = jnp.maximum(h, 0.0)
    o_ref[...] = jax.lax.dot_general(
        h, w2_ref[...], (((1,), (1,)), ((), ())),
        preferred_element_type=jnp.float32) + b2_ref[...]


def kernel(features, points, nuv, ranges, W_lt, b_lt, W1, b1, W2, b2):
    del points, nuv, ranges
    n, d_in = features.shape
    d_out = W_lt.shape[0]
    weight_spec = lambda shape: pl.BlockSpec(shape, lambda i: (0, 0))
    return pl.pallas_call(
        _mlp_kernel,
        grid=(pl.cdiv(n, _BLOCK),),
        in_specs=[
            pl.BlockSpec((_BLOCK, d_in), lambda i: (i, 0)),
            weight_spec((d_out, d_in)),
            weight_spec((1, d_out)),
            weight_spec((d_out, d_out)),
            weight_spec((1, d_out)),
            weight_spec((d_out, d_out)),
            weight_spec((1, d_out)),
        ],
        out_specs=pl.BlockSpec((_BLOCK, d_out), lambda i: (i, 0)),
        out_shape=jax.ShapeDtypeStruct((n, d_out), jnp.float32),
        compiler_params=pltpu.CompilerParams(
            dimension_semantics=("parallel",),
        ),
    )(features, W_lt, b_lt[None, :], W1, b1[None, :], W2, b2[None, :])
